# Initial kernel scaffold; baseline (speedup 1.0000x reference)
import jax
import jax.numpy as jnp
from jax.experimental import pallas as pl


def kernel(memory, node_features, time_w, time_b, timestamps, source_nodes, n_layers):
    raise NotImplementedError("write your pallas kernel here")


# speedup vs baseline: 1.2015x; 1.2015x over previous
import functools

import jax
import jax.numpy as jnp
from jax import lax
from jax.experimental import pallas as pl
from jax.experimental.pallas import tpu as pltpu
from jax.experimental.pallas import tpu_sc as plsc

_C = 128
_NW = 32


def _gather_add(mem, feat, idx):
    n, d = mem.shape
    b = idx.shape[0]
    n_chunks = -(-b // _C)
    slots = -(-n_chunks // _NW)
    last_base = b - _C

    mesh = plsc.VectorSubcoreMesh(core_axis_name="c", subcore_axis_name="s")

    @functools.partial(
        pl.kernel,
        mesh=mesh,
        out_type=jax.ShapeDtypeStruct((b, d), jnp.float32),
        scratch_types=[
            pltpu.VMEM((_C,), jnp.int32),
            pltpu.VMEM((_C, d), jnp.float32),
            pltpu.VMEM((_C, d), jnp.float32),
            pltpu.SemaphoreType.DMA,
            pltpu.SemaphoreType.DMA,
        ],
    )
    def k(mem_hbm, feat_hbm, idx_hbm, out_hbm, idx_v, bufm, buff, sem_a, sem_b):
        wid = lax.axis_index("s") * 2 + lax.axis_index("c")

        def slot_body(s, carry):
            chunk = s * _NW + wid

            @pl.when(chunk < n_chunks)
            def _():
                base = jnp.minimum(chunk * _C, last_base)
                pltpu.sync_copy(idx_hbm.at[pl.ds(base, _C)], idx_v)
                cp_a = pltpu.async_copy(mem_hbm.at[idx_v], bufm, sem_a)
                cp_b = pltpu.async_copy(feat_hbm.at[idx_v], buff, sem_b)
                cp_a.wait()
                cp_b.wait()

                def row_body(r, c2):
                    for c in range(d // 16):
                        sl = pl.ds(c * 16, 16)
                        bufm[r, sl] = bufm[r, sl] + buff[r, sl]
                    return c2

                lax.fori_loop(0, _C, row_body, 0)
                pltpu.sync_copy(bufm, out_hbm.at[pl.ds(base, _C)])

            return carry

        lax.fori_loop(0, slots, slot_body, 0)

    return k(mem, feat, idx)


def kernel(memory, node_features, time_w, time_b, timestamps, source_nodes, n_layers):
    del time_w, time_b, timestamps, n_layers
    return _gather_add(memory, node_features, source_nodes.astype(jnp.int32))


# speedup vs baseline: 1.9897x; 1.6560x over previous
import functools

import jax
import jax.numpy as jnp
from jax import lax
from jax.experimental import pallas as pl
from jax.experimental.pallas import tpu as pltpu
from jax.experimental.pallas import tpu_sc as plsc

_C = 128
_NW = 32


def _gather_add(mem, feat, idx):
    n, d = mem.shape
    b = idx.shape[0]
    n_chunks = -(-b // _C)
    last_base = b - _C
    q, r = divmod(n_chunks, _NW)
    npairs = -(-(q + 1) // 2)

    mesh = plsc.VectorSubcoreMesh(core_axis_name="c", subcore_axis_name="s")

    @functools.partial(
        pl.kernel,
        mesh=mesh,
        out_type=jax.ShapeDtypeStruct((b, d), jnp.float32),
        scratch_types=[
            pltpu.VMEM((_C,), jnp.int32),
            pltpu.VMEM((_C,), jnp.int32),
            pltpu.VMEM((_C, d), jnp.float32),
            pltpu.VMEM((_C, d), jnp.float32),
            pltpu.VMEM((_C, d), jnp.float32),
            pltpu.VMEM((_C, d), jnp.float32),
            pltpu.VMEM((_C, d), jnp.float32),
            pltpu.VMEM((_C, d), jnp.float32),
            pltpu.SemaphoreType.DMA,
            pltpu.SemaphoreType.DMA,
            pltpu.SemaphoreType.DMA,
            pltpu.SemaphoreType.DMA,
        ],
    )
    def k(mem_hbm, feat_hbm, idx_hbm, out_hbm,
          idx0, idx1, bm0, bm1, bf0, bf1, ob0, ob1, ga0, ga1, st0, st1):
        wid = lax.axis_index("s") * 2 + lax.axis_index("c")
        nmine = q + (wid < r)
        s0 = wid * q + jnp.minimum(wid, r)

        ring = ((idx0, bm0, bf0, ob0, ga0, st0),
                (idx1, bm1, bf1, ob1, ga1, st1))

        def chunk_base(lc):
            return jnp.minimum((s0 + lc) * _C, last_base)

        def prefetch(lc, bufs):
            idxv, bm, bf, _, ga, _ = bufs
            base = chunk_base(lc)
            pltpu.sync_copy(idx_hbm.at[pl.ds(base, _C)], idxv)
            pltpu.async_copy(mem_hbm.at[idxv], bm, ga)
            pltpu.async_copy(feat_hbm.at[idxv], bf, ga)

        def step(lc, bufs):
            idxv, bm, bf, ob, ga, st = bufs
            pltpu.make_async_copy(mem_hbm.at[idxv], bm, ga).wait()
            pltpu.make_async_copy(feat_hbm.at[idxv], bf, ga).wait()

            @pl.when(lc >= 2)
            def _():
                pltpu.make_async_copy(ob, out_hbm.at[pl.ds(0, _C)], st).wait()

            def row_body(rr, c2):
                for c in range(d // 16):
                    sl = pl.ds(c * 16, 16)
                    ob[rr, sl] = bm[rr, sl] + bf[rr, sl]
                return c2

            lax.fori_loop(0, _C, row_body, 0)

            pltpu.async_copy(ob, out_hbm.at[pl.ds(chunk_base(lc), _C)], st)

            @pl.when(lc + 2 < nmine)
            def _():
                prefetch(lc + 2, bufs)

        @pl.when(0 < nmine)
        def _():
            prefetch(0, ring[0])

        @pl.when(1 < nmine)
        def _():
            prefetch(1, ring[1])

        def pair_body(g, carry):
            for half in range(2):
                lc = 2 * g + half

                @pl.when(lc < nmine)
                def _():
                    step(lc, ring[half])

            return carry

        lax.fori_loop(0, npairs, pair_body, 0)

        @pl.when(nmine >= 1)
        def _():
            pltpu.make_async_copy(ob0, out_hbm.at[pl.ds(0, _C)], st0).wait()

        @pl.when(nmine >= 2)
        def _():
            pltpu.make_async_copy(ob1, out_hbm.at[pl.ds(0, _C)], st1).wait()

    return k(mem, feat, idx)


def kernel(memory, node_features, time_w, time_b, timestamps, source_nodes, n_layers):
    del time_w, time_b, timestamps, n_layers
    return _gather_add(memory, node_features, source_nodes.astype(jnp.int32))
